# baseline (speedup 1.0000x reference)
import functools

import jax
import jax.numpy as jnp
from jax import lax
from jax.experimental import pallas as pl
from jax.experimental.pallas import tpu as pltpu
from jax.experimental.pallas import tpu_sc as plsc

_BATCH = 16384
_COLS = 512
_PATHWAYS = 64
_GENES = 8

_TC_ROWS = 12288
_SC_ROWS = _BATCH - _TC_ROWS
_TC_BLK = 4096

_NC = 2
_NS = 16
_NW = _NC * _NS
_ROWS_PER_W = _SC_ROWS // _NW
_CHUNK = 128
_NCHUNK = _ROWS_PER_W // _CHUNK


def _sc_body(x_hbm, out_hbm, in0, in1, ou0, ou1, si0, si1, so0, so1):
    wid = lax.axis_index("s") * _NC + lax.axis_index("c")
    rbase = _TC_ROWS + wid * _ROWS_PER_W
    cbase = wid * _ROWS_PER_W
    lane = lax.iota(jnp.int32, 16)

    in_bufs = (in0, in1)
    out_bufs = (ou0, ou1)
    in_sems = (si0, si1)
    out_sems = (so0, so1)

    def start_in(ci, b):
        return pltpu.async_copy(
            x_hbm.at[pl.ds(rbase + ci * _CHUNK, _CHUNK)], in_bufs[b], in_sems[b]
        )

    def one_group(in_v, out_v, r0):
        row_idx = lane + r0
        for p in range(_PATHWAYS):
            g = [
                plsc.load_gather(
                    in_v, [row_idx, jnp.full((16,), 8 * p + k, jnp.int32)]
                )
                for k in range(_GENES)
            ]
            s = ((g[0] + g[1]) + (g[2] + g[3])) + ((g[4] + g[5]) + (g[6] + g[7]))
            out_v[p, pl.ds(r0, 16)] = s * 0.125

    out_copies = {}
    start_in(0, 0)
    for ci in range(_NCHUNK):
        b = ci % 2
        pltpu.make_async_copy(
            x_hbm.at[pl.ds(rbase + ci * _CHUNK, _CHUNK)], in_bufs[b], in_sems[b]
        ).wait()
        if ci + 1 < _NCHUNK:
            start_in(ci + 1, 1 - b)
        if ci >= 2:
            out_copies[ci - 2].wait()

        @plsc.parallel_loop(0, _CHUNK, step=16, unroll=1)
        def _groups(r0):
            one_group(in_bufs[b], out_bufs[b], r0)

        out_copies[ci] = pltpu.async_copy(
            out_bufs[b],
            out_hbm.at[:, pl.ds(cbase + ci * _CHUNK, _CHUNK)],
            out_sems[b],
        )
    for ci in range(max(0, _NCHUNK - 2), _NCHUNK):
        out_copies[ci].wait()


def _sc_call(x):
    mesh = plsc.VectorSubcoreMesh(core_axis_name="c", subcore_axis_name="s")
    run = pl.kernel(
        _sc_body,
        out_type=jax.ShapeDtypeStruct((_PATHWAYS, _SC_ROWS), jnp.float32),
        mesh=mesh,
        scratch_types=[
            pltpu.VMEM((_CHUNK, _COLS), jnp.float32),
            pltpu.VMEM((_CHUNK, _COLS), jnp.float32),
            pltpu.VMEM((_PATHWAYS, _CHUNK), jnp.float32),
            pltpu.VMEM((_PATHWAYS, _CHUNK), jnp.float32),
            pltpu.SemaphoreType.DMA,
            pltpu.SemaphoreType.DMA,
            pltpu.SemaphoreType.DMA,
            pltpu.SemaphoreType.DMA,
        ],
        compiler_params=pltpu.CompilerParams(
            use_tc_tiling_on_sc=True, needs_layout_passes=False
        ),
    )
    return run(x)


def _tc_body(x_ref, w_ref, o_ref):
    o_ref[...] = jax.lax.dot_general(
        w_ref[...], x_ref[...], (((0,), (1,)), ((), ())),
        preferred_element_type=jnp.float32,
    )


def _tc_call(x, w):
    return pl.pallas_call(
        _tc_body,
        grid=(_TC_ROWS // _TC_BLK,),
        in_specs=[
            pl.BlockSpec((_TC_BLK, _COLS), lambda i: (i, 0)),
            pl.BlockSpec((_COLS, _PATHWAYS), lambda i: (0, 0)),
        ],
        out_specs=pl.BlockSpec((_PATHWAYS, _TC_BLK), lambda i: (0, i)),
        out_shape=jax.ShapeDtypeStruct((_PATHWAYS, _TC_ROWS), jnp.float32),
    )(x, w)


@jax.jit
def kernel(geneset_features):
    w = jnp.repeat(jnp.eye(_PATHWAYS, dtype=jnp.float32), _GENES, axis=0) * (
        1.0 / _GENES
    )
    sc_t = _sc_call(geneset_features)
    tc_t = _tc_call(geneset_features, w)
    out_t = jnp.concatenate([tc_t, sc_t], axis=1)
    return out_t.T


# speedup vs baseline: 1.6869x; 1.6869x over previous
import functools

import jax
import jax.numpy as jnp
from jax import lax
from jax.experimental import pallas as pl
from jax.experimental.pallas import tpu as pltpu
from jax.experimental.pallas import tpu_sc as plsc

_BATCH = 16384
_COLS = 512
_PATHWAYS = 64
_GENES = 8

_TC_ROWS = 12288
_SC_ROWS = _BATCH - _TC_ROWS
_TC_BLK = 4096

_NC = 2
_NS = 16
_NW = _NC * _NS
_ROWS_PER_W = _SC_ROWS // _NW
_CHUNK = 128
_NCHUNK = _ROWS_PER_W // _CHUNK


def _sc_body(x_hbm, out_hbm, in0, in1, ou0, ou1, si0, si1, so0, so1):
    wid = lax.axis_index("s") * _NC + lax.axis_index("c")
    rbase = _TC_ROWS + wid * _ROWS_PER_W
    cbase = wid * _ROWS_PER_W
    lane = lax.iota(jnp.int32, 16)

    in_bufs = (in0, in1)
    out_bufs = (ou0, ou1)
    in_sems = (si0, si1)
    out_sems = (so0, so1)

    def start_in(ci, b):
        return pltpu.async_copy(
            x_hbm.at[pl.ds(rbase + ci * _CHUNK, _CHUNK)], in_bufs[b], in_sems[b]
        )

    col_idx = [lane * 8 + (128 * j + g) for j in range(4) for g in range(_GENES)]

    def one_row(in_v, out_v, r):
        row_idx = jnp.full((16,), r, jnp.int32)
        for j in range(4):
            g = [plsc.load_gather(in_v, [row_idx, col_idx[8 * j + k]])
                 for k in range(_GENES)]
            s = ((g[0] + g[1]) + (g[2] + g[3])) + ((g[4] + g[5]) + (g[6] + g[7]))
            plsc.store_scatter(out_v, [lane + 16 * j, row_idx], s * 0.125)

    out_copies = {}
    start_in(0, 0)
    for ci in range(_NCHUNK):
        b = ci % 2
        pltpu.make_async_copy(
            x_hbm.at[pl.ds(rbase + ci * _CHUNK, _CHUNK)], in_bufs[b], in_sems[b]
        ).wait()
        if ci + 1 < _NCHUNK:
            start_in(ci + 1, 1 - b)
        if ci >= 2:
            out_copies[ci - 2].wait()

        @plsc.parallel_loop(0, _CHUNK, step=1, unroll=4)
        def _rows(r):
            one_row(in_bufs[b], out_bufs[b], r)

        out_copies[ci] = pltpu.async_copy(
            out_bufs[b],
            out_hbm.at[:, pl.ds(cbase + ci * _CHUNK, _CHUNK)],
            out_sems[b],
        )
    for ci in range(max(0, _NCHUNK - 2), _NCHUNK):
        out_copies[ci].wait()


def _sc_call(x):
    mesh = plsc.VectorSubcoreMesh(core_axis_name="c", subcore_axis_name="s")
    run = pl.kernel(
        _sc_body,
        out_type=jax.ShapeDtypeStruct((_PATHWAYS, _SC_ROWS), jnp.float32),
        mesh=mesh,
        scratch_types=[
            pltpu.VMEM((_CHUNK, _COLS), jnp.float32),
            pltpu.VMEM((_CHUNK, _COLS), jnp.float32),
            pltpu.VMEM((_PATHWAYS, _CHUNK), jnp.float32),
            pltpu.VMEM((_PATHWAYS, _CHUNK), jnp.float32),
            pltpu.SemaphoreType.DMA,
            pltpu.SemaphoreType.DMA,
            pltpu.SemaphoreType.DMA,
            pltpu.SemaphoreType.DMA,
        ],
        compiler_params=pltpu.CompilerParams(
            use_tc_tiling_on_sc=True, needs_layout_passes=False
        ),
    )
    return run(x)


def _tc_body(x_ref, w_ref, o_ref):
    o_ref[...] = jax.lax.dot_general(
        w_ref[...], x_ref[...], (((0,), (1,)), ((), ())),
        preferred_element_type=jnp.float32,
    )


def _tc_call(x, w):
    return pl.pallas_call(
        _tc_body,
        grid=(_TC_ROWS // _TC_BLK,),
        in_specs=[
            pl.BlockSpec((_TC_BLK, _COLS), lambda i: (i, 0)),
            pl.BlockSpec((_COLS, _PATHWAYS), lambda i: (0, 0)),
        ],
        out_specs=pl.BlockSpec((_PATHWAYS, _TC_BLK), lambda i: (0, i)),
        out_shape=jax.ShapeDtypeStruct((_PATHWAYS, _TC_ROWS), jnp.float32),
    )(x, w)


@jax.jit
def kernel(geneset_features):
    w = jnp.repeat(jnp.eye(_PATHWAYS, dtype=jnp.float32), _GENES, axis=0) * (
        1.0 / _GENES
    )
    sc_t = _sc_call(geneset_features)
    tc_t = _tc_call(geneset_features, w)
    out_t = jnp.concatenate([tc_t, sc_t], axis=1)
    return out_t.T
